# Initial kernel scaffold; baseline (speedup 1.0000x reference)
import functools

import jax
import jax.numpy as jnp
from jax import lax
from jax.experimental import pallas as pl
from jax.experimental.pallas import tpu as pltpu
from jax.experimental.pallas import tpu_sc as plsc

N = 10000
E = 320000
IN_DIM = 128
HID = 128

NCORE = 2
NTILE = 16
NWORK = NCORE * NTILE
ROWS_PER_TILE = N // NTILE
DEG_EDGES_PER_WORKER = E // NWORK
AGG_EDGES_PER_TILE = E // NTILE
KB = 80
NBATCH = AGG_EDGES_PER_TILE // KB

_MESH = plsc.VectorSubcoreMesh(core_axis_name="c", subcore_axis_name="s")


def _deg_body(dst_hbm, hist_hbm, dst_v, hist_v, sem):
    cid = lax.axis_index("c")
    sid = lax.axis_index("s")
    w = cid * NTILE + sid
    zeros = jnp.zeros((16,), jnp.float32)

    def _zero(i, _):
        hist_v[pl.ds(i * 16, 16)] = zeros
        return 0

    lax.fori_loop(0, N // 16, _zero, 0, unroll=4)
    pltpu.async_copy(
        dst_hbm.at[pl.ds(w * DEG_EDGES_PER_WORKER, DEG_EDGES_PER_WORKER)],
        dst_v, sem).wait()
    ones = jnp.ones((16,), jnp.float32)

    def _acc(i, _):
        idx = dst_v[pl.ds(i * 16, 16)]
        plsc.addupdate_scatter(hist_v, [idx], ones)
        return 0

    lax.fori_loop(0, DEG_EDGES_PER_WORKER // 16, _acc, 0, unroll=4)
    pltpu.async_copy(hist_v, hist_hbm.at[w], sem).wait()


_deg_call = pl.kernel(
    _deg_body,
    out_type=jax.ShapeDtypeStruct((NWORK, N), jnp.float32),
    mesh=_MESH,
    scratch_types=[
        pltpu.VMEM((DEG_EDGES_PER_WORKER,), jnp.int32),
        pltpu.VMEM((N,), jnp.float32),
        pltpu.SemaphoreType.DMA,
    ],
)


def _agg_body(h0_hbm, h1_hbm, src_hbm, dst_hbm, o0_hbm, o1_hbm,
              src_v, dst_v, rows_v, acc, sem):
    cid = lax.axis_index("c")
    sid = lax.axis_index("s")
    base_e = sid * AGG_EDGES_PER_TILE
    rs = sid * ROWS_PER_TILE
    pltpu.async_copy(src_hbm.at[pl.ds(base_e, AGG_EDGES_PER_TILE)], src_v,
                     sem).wait()
    pltpu.async_copy(dst_hbm.at[pl.ds(sid * NBATCH, NBATCH)], dst_v,
                     sem).wait()

    def _run(h_hbm, out_hbm):
        pltpu.sync_copy(h_hbm.at[pl.ds(rs, ROWS_PER_TILE)],
                        acc.at[pl.ds(rs, ROWS_PER_TILE)])
        plsc.subcore_barrier()

        def _batch(j, _):
            gidx = src_v.at[pl.ds(j * KB, KB)]
            pltpu.async_copy(h_hbm.at[gidx], rows_v, sem).wait()
            pltpu.sync_copy(rows_v, acc.at[dst_v.at[j]], add=True)
            return 0

        lax.fori_loop(0, NBATCH, _batch, 0)
        plsc.subcore_barrier()
        pltpu.sync_copy(acc.at[pl.ds(rs, ROWS_PER_TILE)],
                        out_hbm.at[pl.ds(rs, ROWS_PER_TILE)])

    @pl.when(cid == 0)
    def _():
        _run(h0_hbm, o0_hbm)

    @pl.when(cid == 1)
    def _():
        _run(h1_hbm, o1_hbm)


def _make_agg(d):
    return pl.kernel(
        _agg_body,
        out_type=(jax.ShapeDtypeStruct((N, d), jnp.float32),
                  jax.ShapeDtypeStruct((N, d), jnp.float32)),
        mesh=_MESH,
        scratch_types=[
            pltpu.VMEM((AGG_EDGES_PER_TILE,), jnp.int32),
            pltpu.VMEM((NBATCH, KB), jnp.int32),
            pltpu.VMEM((KB, d), jnp.float32),
            pltpu.VMEM_SHARED((N, d), jnp.float32),
            pltpu.SemaphoreType.DMA,
        ],
    )


_agg128 = _make_agg(2 * HID // 2)
_agg64 = _make_agg(HID // 2)


def _tca_body(x_ref, w1_ref, hist_ref, o0_ref, o1_ref, dinv_ref):
    y = lax.dot_general(x_ref[...], w1_ref[...], (((1,), (1,)), ((), ())),
                        preferred_element_type=jnp.float32)
    deg = jnp.sum(hist_ref[...], axis=0) + 1.0
    dinv = lax.rsqrt(deg)
    o0_ref[...] = y[:, :HID] * dinv[:, None]
    o1_ref[...] = y[:, HID:] * dinv[:, None]
    dinv_ref[...] = dinv[None, :]


def _tcb_body(a0_ref, a1_ref, h0_ref, h1_ref, dinv_ref, b1_ref, g1_ref,
              be1_ref, w2_ref, o0_ref, o1_ref):
    dv = dinv_ref[...][0][:, None]
    t = jnp.concatenate([a0_ref[...], a1_ref[...]], axis=1) * dv + b1_ref[...]
    mu = jnp.mean(t, axis=1, keepdims=True)
    c = t - mu
    var = jnp.mean(c * c, axis=1, keepdims=True)
    ln = c * lax.rsqrt(var + 1e-5) * g1_ref[...] + be1_ref[...]
    ge = 0.5 * ln * (1.0 + lax.erf(ln * 0.7071067811865476))
    y = lax.dot_general(ge, w2_ref[...], (((1,), (1,)), ((), ())),
                        preferred_element_type=jnp.float32) * dv
    o0_ref[...] = y[:, :HID // 2]
    o1_ref[...] = y[:, HID // 2:]


def _tcc_body(a0_ref, a1_ref, dinv_ref, b2_ref, g2_ref, be2_ref, o_ref):
    dv = dinv_ref[...][0][:, None]
    t = jnp.concatenate([a0_ref[...], a1_ref[...]], axis=1) * dv + b2_ref[...]
    mu = jnp.mean(t, axis=1, keepdims=True)
    c = t - mu
    var = jnp.mean(c * c, axis=1, keepdims=True)
    o_ref[...] = c * lax.rsqrt(var + 1e-5) * g2_ref[...] + be2_ref[...]


_BM = 1000
_GRID = N // _BM


def _row_spec(d):
    return pl.BlockSpec((_BM, d), lambda i: (i, 0))


def _full_spec(shape):
    return pl.BlockSpec(shape, lambda i: tuple(0 for _ in shape))


_DINV_SPEC = pl.BlockSpec((1, _BM), lambda i: (0, i))

_tca_call = pl.pallas_call(
    _tca_body,
    grid=(_GRID,),
    in_specs=[_row_spec(IN_DIM), _full_spec((2 * HID, IN_DIM)),
              pl.BlockSpec((NWORK, _BM), lambda i: (0, i))],
    out_specs=[_row_spec(HID), _row_spec(HID), _DINV_SPEC],
    out_shape=[jax.ShapeDtypeStruct((N, HID), jnp.float32),
               jax.ShapeDtypeStruct((N, HID), jnp.float32),
               jax.ShapeDtypeStruct((1, N), jnp.float32)],
)

_tcb_call = pl.pallas_call(
    _tcb_body,
    grid=(_GRID,),
    in_specs=[_row_spec(HID), _row_spec(HID), _row_spec(HID), _row_spec(HID),
              _DINV_SPEC, _full_spec((1, 2 * HID)), _full_spec((1, 2 * HID)),
              _full_spec((1, 2 * HID)), _full_spec((HID, 2 * HID))],
    out_specs=[_row_spec(HID // 2), _row_spec(HID // 2)],
    out_shape=[jax.ShapeDtypeStruct((N, HID // 2), jnp.float32),
               jax.ShapeDtypeStruct((N, HID // 2), jnp.float32)],
)

_tcc_call = pl.pallas_call(
    _tcc_body,
    grid=(_GRID,),
    in_specs=[_row_spec(HID // 2), _row_spec(HID // 2), _DINV_SPEC,
              _full_spec((1, HID)), _full_spec((1, HID)),
              _full_spec((1, HID))],
    out_specs=_row_spec(HID),
    out_shape=jax.ShapeDtypeStruct((N, HID), jnp.float32),
)


def kernel(x, edge_index, W1, b1, g1, be1, W2, b2, g2, be2):
    src = edge_index[0]
    dst = edge_index[1]
    dst2d = dst.reshape(E // KB, KB)
    hist = _deg_call(dst)
    h0, h1, dinv = _tca_call(x, W1, hist)
    a0, a1 = _agg128(h0, h1, src, dst2d)
    p0, p1 = _tcb_call(a0, a1, h0, h1, dinv, b1.reshape(1, -1),
                       g1.reshape(1, -1), be1.reshape(1, -1), W2)
    q0, q1 = _agg64(p0, p1, src, dst2d)
    return _tcc_call(q0, q1, dinv, b2.reshape(1, -1), g2.reshape(1, -1),
                     be2.reshape(1, -1))


# speedup vs baseline: 11.6273x; 11.6273x over previous
import jax
import jax.numpy as jnp
from jax import lax
from jax.experimental import pallas as pl
from jax.experimental.pallas import tpu as pltpu
from jax.experimental.pallas import tpu_sc as plsc

N = 10000
E = 320000
IN_DIM = 128
HID = 128
D = 64

NCORE = 2
NTILE = 16
NWORK = NCORE * NTILE
ROWS_PER_TILE = 624
TAIL_ROWS = N - NTILE * ROWS_PER_TILE
DEG_EDGES_PER_WORKER = E // NWORK
AGG_EDGES_PER_TILE = E // NTILE
KB = 80
NBATCH = AGG_EDGES_PER_TILE // KB

_MESH = plsc.VectorSubcoreMesh(core_axis_name="c", subcore_axis_name="s")


def _deg_body(dst_hbm, hist_hbm, dst_v, hist_v, sem):
    cid = lax.axis_index("c")
    sid = lax.axis_index("s")
    w = cid * NTILE + sid
    zeros = jnp.zeros((16,), jnp.float32)

    def _zero(i, _):
        hist_v[pl.ds(i * 16, 16)] = zeros
        return 0

    lax.fori_loop(0, N // 16, _zero, 0, unroll=4)
    pltpu.async_copy(
        dst_hbm.at[pl.ds(w * DEG_EDGES_PER_WORKER, DEG_EDGES_PER_WORKER)],
        dst_v, sem).wait()
    ones = jnp.ones((16,), jnp.float32)

    def _acc(i, _):
        idx = dst_v[pl.ds(i * 16, 16)]
        plsc.addupdate_scatter(hist_v, [idx], ones)
        return 0

    lax.fori_loop(0, DEG_EDGES_PER_WORKER // 16, _acc, 0, unroll=4)
    pltpu.async_copy(hist_v, hist_hbm.at[pl.ds(w * N, N)], sem).wait()


_deg_call = pl.kernel(
    _deg_body,
    out_type=jax.ShapeDtypeStruct((NWORK * N,), jnp.float32),
    mesh=_MESH,
    compiler_params=pltpu.CompilerParams(needs_layout_passes=False),
    scratch_types=[
        pltpu.VMEM((DEG_EDGES_PER_WORKER,), jnp.int32),
        pltpu.VMEM((N,), jnp.float32),
        pltpu.SemaphoreType.DMA,
    ],
)


def _agg_body(h0_hbm, h1_hbm, src_hbm, dst_hbm, o0_hbm, o1_hbm,
              src_v, dst_v, rows_v, acc, sem):
    cid = lax.axis_index("c")
    sid = lax.axis_index("s")
    base_e = sid * AGG_EDGES_PER_TILE
    rs = sid * ROWS_PER_TILE
    pltpu.async_copy(src_hbm.at[pl.ds(base_e, AGG_EDGES_PER_TILE)], src_v,
                     sem).wait()
    pltpu.async_copy(dst_hbm.at[sid], dst_v, sem).wait()

    tail = NTILE * ROWS_PER_TILE

    def _run(h_hbm, out_hbm):
        pltpu.sync_copy(h_hbm.at[pl.ds(rs, ROWS_PER_TILE)],
                        acc.at[pl.ds(rs, ROWS_PER_TILE)])

        @pl.when(sid == 0)
        def _():
            pltpu.sync_copy(h_hbm.at[pl.ds(tail, TAIL_ROWS)],
                            acc.at[pl.ds(tail, TAIL_ROWS)])

        plsc.subcore_barrier()

        def _batch(j, _):
            gidx = src_v.at[pl.ds(j * KB, KB)]
            pltpu.async_copy(h_hbm.at[gidx], rows_v, sem).wait()
            pltpu.sync_copy(rows_v, acc.at[dst_v.at[j]], add=True)
            return 0

        lax.fori_loop(0, NBATCH, _batch, 0)
        plsc.subcore_barrier()
        pltpu.sync_copy(acc.at[pl.ds(rs, ROWS_PER_TILE)],
                        out_hbm.at[pl.ds(rs, ROWS_PER_TILE)])

        @pl.when(sid == 0)
        def _():
            pltpu.sync_copy(acc.at[pl.ds(tail, TAIL_ROWS)],
                            out_hbm.at[pl.ds(tail, TAIL_ROWS)])

    @pl.when(cid == 0)
    def _():
        _run(h0_hbm, o0_hbm)

    @pl.when(cid == 1)
    def _():
        _run(h1_hbm, o1_hbm)


_agg_call = pl.kernel(
    _agg_body,
    out_type=(jax.ShapeDtypeStruct((N, D), jnp.float32),
              jax.ShapeDtypeStruct((N, D), jnp.float32)),
    mesh=_MESH,
    compiler_params=pltpu.CompilerParams(use_tc_tiling_on_sc=False),
    scratch_types=[
        pltpu.VMEM((AGG_EDGES_PER_TILE,), jnp.int32),
        pltpu.VMEM((NBATCH, KB), jnp.int32),
        pltpu.VMEM((KB, D), jnp.float32),
        pltpu.VMEM_SHARED((N, D), jnp.float32),
        pltpu.SemaphoreType.DMA,
    ],
)


def _dinv_body(hist_ref, dinv_ref):
    deg = jnp.sum(hist_ref[...], axis=0) + 1.0
    dinv_ref[...] = lax.rsqrt(deg)[:, None]


def _tca_body(x_ref, w1_ref, dinv_ref, o0_ref, o1_ref, o2_ref, o3_ref):
    y = lax.dot_general(x_ref[...], w1_ref[...], (((1,), (1,)), ((), ())),
                        preferred_element_type=jnp.float32)
    dv = dinv_ref[...]
    o0_ref[...] = y[:, 0 * D:1 * D] * dv
    o1_ref[...] = y[:, 1 * D:2 * D] * dv
    o2_ref[...] = y[:, 2 * D:3 * D] * dv
    o3_ref[...] = y[:, 3 * D:4 * D] * dv


def _tcb_body(a0_ref, a1_ref, a2_ref, a3_ref, dinv_ref, b1_ref, g1_ref,
              be1_ref, w2_ref, o0_ref, o1_ref):
    dv = dinv_ref[...]
    t = jnp.concatenate(
        [a0_ref[...], a1_ref[...], a2_ref[...], a3_ref[...]], axis=1
    ) * dv + b1_ref[...]
    mu = jnp.mean(t, axis=1, keepdims=True)
    c = t - mu
    var = jnp.mean(c * c, axis=1, keepdims=True)
    ln = c * lax.rsqrt(var + 1e-5) * g1_ref[...] + be1_ref[...]
    ge = 0.5 * ln * (1.0 + lax.erf(ln * 0.7071067811865476))
    y = lax.dot_general(ge, w2_ref[...], (((1,), (1,)), ((), ())),
                        preferred_element_type=jnp.float32) * dv
    o0_ref[...] = y[:, :D]
    o1_ref[...] = y[:, D:]


def _tcc_body(a0_ref, a1_ref, dinv_ref, b2_ref, g2_ref, be2_ref, o_ref):
    dv = dinv_ref[...]
    t = jnp.concatenate([a0_ref[...], a1_ref[...]], axis=1) * dv + b2_ref[...]
    mu = jnp.mean(t, axis=1, keepdims=True)
    c = t - mu
    var = jnp.mean(c * c, axis=1, keepdims=True)
    o_ref[...] = c * lax.rsqrt(var + 1e-5) * g2_ref[...] + be2_ref[...]


_BM = 1000
_GRID = N // _BM


def _row_spec(d):
    return pl.BlockSpec((_BM, d), lambda i: (i, 0))


def _full_spec(shape):
    return pl.BlockSpec(shape, lambda i: tuple(0 for _ in shape))


_DINV_SPEC = pl.BlockSpec((_BM, 1), lambda i: (i, 0))

_dinv_call = pl.pallas_call(
    _dinv_body,
    out_shape=jax.ShapeDtypeStruct((N, 1), jnp.float32),
)

_tca_call = pl.pallas_call(
    _tca_body,
    grid=(_GRID,),
    in_specs=[_row_spec(IN_DIM), _full_spec((2 * HID, IN_DIM)), _DINV_SPEC],
    out_specs=[_row_spec(D)] * 4,
    out_shape=[jax.ShapeDtypeStruct((N, D), jnp.float32)] * 4,
)

_tcb_call = pl.pallas_call(
    _tcb_body,
    grid=(_GRID,),
    in_specs=[_row_spec(D), _row_spec(D), _row_spec(D), _row_spec(D),
              _DINV_SPEC, _full_spec((1, 2 * HID)), _full_spec((1, 2 * HID)),
              _full_spec((1, 2 * HID)), _full_spec((HID, 2 * HID))],
    out_specs=[_row_spec(D), _row_spec(D)],
    out_shape=[jax.ShapeDtypeStruct((N, D), jnp.float32),
               jax.ShapeDtypeStruct((N, D), jnp.float32)],
)

_tcc_call = pl.pallas_call(
    _tcc_body,
    grid=(_GRID,),
    in_specs=[_row_spec(D), _row_spec(D), _DINV_SPEC,
              _full_spec((1, HID)), _full_spec((1, HID)),
              _full_spec((1, HID))],
    out_specs=_row_spec(HID),
    out_shape=jax.ShapeDtypeStruct((N, HID), jnp.float32),
)


def kernel(x, edge_index, W1, b1, g1, be1, W2, b2, g2, be2):
    src = edge_index[0]
    dst = edge_index[1]
    dst3d = dst.reshape(NTILE, NBATCH, KB)
    hist = _deg_call(dst).reshape(NWORK, N)
    dinv = _dinv_call(hist)
    h0, h1, h2, h3 = _tca_call(x, W1, dinv)
    a0, a1 = _agg_call(h0, h1, src, dst3d)
    a2, a3 = _agg_call(h2, h3, src, dst3d)
    p0, p1 = _tcb_call(a0, a1, a2, a3, dinv, b1.reshape(1, -1),
                       g1.reshape(1, -1), be1.reshape(1, -1), W2)
    q0, q1 = _agg_call(p0, p1, src, dst3d)
    return _tcc_call(q0, q1, dinv, b2.reshape(1, -1), g2.reshape(1, -1),
                     be2.reshape(1, -1))


# speedup vs baseline: 19.2359x; 1.6544x over previous
import jax
import jax.numpy as jnp
from jax import lax
from jax.experimental import pallas as pl
from jax.experimental.pallas import tpu as pltpu
from jax.experimental.pallas import tpu_sc as plsc

N = 10000
E = 320000
IN_DIM = 128
HID = 128
D = 64

NCORE = 2
NTILE = 16
NWORK = NCORE * NTILE
ROWS_PER_TILE = 624
TAIL_ROWS = N - NTILE * ROWS_PER_TILE
DEG_EDGES_PER_WORKER = E // NWORK
AGG_EDGES_PER_TILE = E // NTILE
KB = 80
NBATCH = AGG_EDGES_PER_TILE // KB

_MESH = plsc.VectorSubcoreMesh(core_axis_name="c", subcore_axis_name="s")


def _deg_body(dst_hbm, hist_hbm, dst_v, hist_v, sem):
    cid = lax.axis_index("c")
    sid = lax.axis_index("s")
    w = cid * NTILE + sid
    zeros = jnp.zeros((16,), jnp.float32)

    def _zero(i, _):
        hist_v[pl.ds(i * 16, 16)] = zeros
        return 0

    lax.fori_loop(0, N // 16, _zero, 0, unroll=4)
    pltpu.async_copy(
        dst_hbm.at[pl.ds(w * DEG_EDGES_PER_WORKER, DEG_EDGES_PER_WORKER)],
        dst_v, sem).wait()
    ones = jnp.ones((16,), jnp.float32)

    def _acc(i, _):
        idx = dst_v[pl.ds(i * 16, 16)]
        plsc.addupdate_scatter(hist_v, [idx], ones)
        return 0

    lax.fori_loop(0, DEG_EDGES_PER_WORKER // 16, _acc, 0, unroll=4)
    pltpu.async_copy(hist_v, hist_hbm.at[pl.ds(w * N, N)], sem).wait()


_deg_call = pl.kernel(
    _deg_body,
    out_type=jax.ShapeDtypeStruct((NWORK * N,), jnp.float32),
    mesh=_MESH,
    compiler_params=pltpu.CompilerParams(needs_layout_passes=False),
    scratch_types=[
        pltpu.VMEM((DEG_EDGES_PER_WORKER,), jnp.int32),
        pltpu.VMEM((N,), jnp.float32),
        pltpu.SemaphoreType.DMA,
    ],
)


def _agg_body(h0_hbm, h1_hbm, src_hbm, dst_hbm, o0_hbm, o1_hbm,
              src_v, dst_v, r0_v, r1_v, acc, sem0, sem1):
    cid = lax.axis_index("c")
    sid = lax.axis_index("s")
    base_e = sid * AGG_EDGES_PER_TILE
    rs = sid * ROWS_PER_TILE
    pltpu.async_copy(src_hbm.at[pl.ds(base_e, AGG_EDGES_PER_TILE)], src_v,
                     sem0).wait()
    pltpu.async_copy(dst_hbm.at[sid], dst_v, sem0).wait()

    tail = NTILE * ROWS_PER_TILE

    def _run(h_hbm, out_hbm):
        pltpu.sync_copy(h_hbm.at[pl.ds(rs, ROWS_PER_TILE)],
                        acc.at[pl.ds(rs, ROWS_PER_TILE)])

        @pl.when(sid == 0)
        def _():
            pltpu.sync_copy(h_hbm.at[pl.ds(tail, TAIL_ROWS)],
                            acc.at[pl.ds(tail, TAIL_ROWS)])

        plsc.subcore_barrier()

        def _gather(j, buf, sem):
            return pltpu.make_async_copy(
                h_hbm.at[src_v.at[pl.ds(j * KB, KB)]], buf, sem)

        _gather(0, r0_v, sem0).start()

        def _batch2(t, _):
            j0 = 2 * t
            _gather(j0 + 1, r1_v, sem1).start()
            _gather(j0, r0_v, sem0).wait()
            pltpu.sync_copy(r0_v, acc.at[dst_v.at[j0]], add=True)

            @pl.when(t + 1 < NBATCH // 2)
            def _():
                _gather(j0 + 2, r0_v, sem0).start()

            _gather(j0 + 1, r1_v, sem1).wait()
            pltpu.sync_copy(r1_v, acc.at[dst_v.at[j0 + 1]], add=True)
            return 0

        lax.fori_loop(0, NBATCH // 2, _batch2, 0)
        plsc.subcore_barrier()
        pltpu.sync_copy(acc.at[pl.ds(rs, ROWS_PER_TILE)],
                        out_hbm.at[pl.ds(rs, ROWS_PER_TILE)])

        @pl.when(sid == 0)
        def _():
            pltpu.sync_copy(acc.at[pl.ds(tail, TAIL_ROWS)],
                            out_hbm.at[pl.ds(tail, TAIL_ROWS)])

    @pl.when(cid == 0)
    def _():
        _run(h0_hbm, o0_hbm)

    @pl.when(cid == 1)
    def _():
        _run(h1_hbm, o1_hbm)


_agg_call = pl.kernel(
    _agg_body,
    out_type=(jax.ShapeDtypeStruct((N, D), jnp.float32),
              jax.ShapeDtypeStruct((N, D), jnp.float32)),
    mesh=_MESH,
    compiler_params=pltpu.CompilerParams(use_tc_tiling_on_sc=False),
    scratch_types=[
        pltpu.VMEM((AGG_EDGES_PER_TILE,), jnp.int32),
        pltpu.VMEM((NBATCH, KB), jnp.int32),
        pltpu.VMEM((KB, D), jnp.float32),
        pltpu.VMEM((KB, D), jnp.float32),
        pltpu.VMEM_SHARED((N, D), jnp.float32),
        pltpu.SemaphoreType.DMA,
        pltpu.SemaphoreType.DMA,
    ],
)


def _dinv_body(hist_ref, dinv_ref):
    deg = jnp.sum(hist_ref[...], axis=0) + 1.0
    dinv_ref[...] = lax.rsqrt(deg)[:, None]


def _tca_body(x_ref, w1_ref, dinv_ref, o0_ref, o1_ref, o2_ref, o3_ref):
    y = lax.dot_general(x_ref[...], w1_ref[...], (((1,), (1,)), ((), ())),
                        preferred_element_type=jnp.float32)
    dv = dinv_ref[...]
    o0_ref[...] = y[:, 0 * D:1 * D] * dv
    o1_ref[...] = y[:, 1 * D:2 * D] * dv
    o2_ref[...] = y[:, 2 * D:3 * D] * dv
    o3_ref[...] = y[:, 3 * D:4 * D] * dv


def _tcb_body(a0_ref, a1_ref, a2_ref, a3_ref, dinv_ref, b1_ref, g1_ref,
              be1_ref, w2_ref, o0_ref, o1_ref):
    dv = dinv_ref[...]
    t = jnp.concatenate(
        [a0_ref[...], a1_ref[...], a2_ref[...], a3_ref[...]], axis=1
    ) * dv + b1_ref[...]
    mu = jnp.mean(t, axis=1, keepdims=True)
    c = t - mu
    var = jnp.mean(c * c, axis=1, keepdims=True)
    ln = c * lax.rsqrt(var + 1e-5) * g1_ref[...] + be1_ref[...]
    ge = 0.5 * ln * (1.0 + lax.erf(ln * 0.7071067811865476))
    y = lax.dot_general(ge, w2_ref[...], (((1,), (1,)), ((), ())),
                        preferred_element_type=jnp.float32) * dv
    o0_ref[...] = y[:, :D]
    o1_ref[...] = y[:, D:]


def _tcc_body(a0_ref, a1_ref, dinv_ref, b2_ref, g2_ref, be2_ref, o_ref):
    dv = dinv_ref[...]
    t = jnp.concatenate([a0_ref[...], a1_ref[...]], axis=1) * dv + b2_ref[...]
    mu = jnp.mean(t, axis=1, keepdims=True)
    c = t - mu
    var = jnp.mean(c * c, axis=1, keepdims=True)
    o_ref[...] = c * lax.rsqrt(var + 1e-5) * g2_ref[...] + be2_ref[...]


_BM = 1000
_GRID = N // _BM


def _row_spec(d):
    return pl.BlockSpec((_BM, d), lambda i: (i, 0))


def _full_spec(shape):
    return pl.BlockSpec(shape, lambda i: tuple(0 for _ in shape))


_DINV_SPEC = pl.BlockSpec((_BM, 1), lambda i: (i, 0))

_dinv_call = pl.pallas_call(
    _dinv_body,
    out_shape=jax.ShapeDtypeStruct((N, 1), jnp.float32),
)

_tca_call = pl.pallas_call(
    _tca_body,
    grid=(_GRID,),
    in_specs=[_row_spec(IN_DIM), _full_spec((2 * HID, IN_DIM)), _DINV_SPEC],
    out_specs=[_row_spec(D)] * 4,
    out_shape=[jax.ShapeDtypeStruct((N, D), jnp.float32)] * 4,
)

_tcb_call = pl.pallas_call(
    _tcb_body,
    grid=(_GRID,),
    in_specs=[_row_spec(D), _row_spec(D), _row_spec(D), _row_spec(D),
              _DINV_SPEC, _full_spec((1, 2 * HID)), _full_spec((1, 2 * HID)),
              _full_spec((1, 2 * HID)), _full_spec((HID, 2 * HID))],
    out_specs=[_row_spec(D), _row_spec(D)],
    out_shape=[jax.ShapeDtypeStruct((N, D), jnp.float32),
               jax.ShapeDtypeStruct((N, D), jnp.float32)],
)

_tcc_call = pl.pallas_call(
    _tcc_body,
    grid=(_GRID,),
    in_specs=[_row_spec(D), _row_spec(D), _DINV_SPEC,
              _full_spec((1, HID)), _full_spec((1, HID)),
              _full_spec((1, HID))],
    out_specs=_row_spec(HID),
    out_shape=jax.ShapeDtypeStruct((N, HID), jnp.float32),
)


def kernel(x, edge_index, W1, b1, g1, be1, W2, b2, g2, be2):
    src = edge_index[0]
    dst = edge_index[1]
    dst3d = dst.reshape(NTILE, NBATCH, KB)
    hist = _deg_call(dst).reshape(NWORK, N)
    dinv = _dinv_call(hist)
    h0, h1, h2, h3 = _tca_call(x, W1, dinv)
    a0, a1 = _agg_call(h0, h1, src, dst3d)
    a2, a3 = _agg_call(h2, h3, src, dst3d)
    p0, p1 = _tcb_call(a0, a1, a2, a3, dinv, b1.reshape(1, -1),
                       g1.reshape(1, -1), be1.reshape(1, -1), W2)
    q0, q1 = _agg_call(p0, p1, src, dst3d)
    return _tcc_call(q0, q1, dinv, b2.reshape(1, -1), g2.reshape(1, -1),
                     be2.reshape(1, -1))
